# baseline (speedup 1.0000x reference)
import jax
import jax.numpy as jnp
from jax.experimental import pallas as pl
from jax.experimental.pallas import tpu as pltpu

_F = 26
_D = 100
_DP = 104
_K = 16
_GK = _F * _K
_GKA = 424
_TP = _F * _DP


def _ffm_block(x_ref, vrt_ref, o_ref, e_ref):
    x = x_ref[...]

    for f in range(_F):
        xf = x[:, f * _D:(f + 1) * _D]
        e_ref[f] = jax.lax.dot_general(
            vrt_ref[f], xf, (((1,), (1,)), ((), ())),
            preferred_element_type=jnp.float32)

    lin = None
    for f in range(_F):
        lrow = e_ref[f, _GK:_GK + 1, :]
        lin = lrow if lin is None else lin + lrow

    acc = None
    for i in range(_F - 1):
        ei = e_ref[i]
        for j in range(i + 1, _F):
            prod = ei[j * _K:(j + 1) * _K, :] * e_ref[j, i * _K:(i + 1) * _K, :]
            acc = prod if acc is None else acc + prod

    o_ref[...] = lin + jnp.sum(acc, axis=0, keepdims=True)


def kernel(inputs, w, v):
    b = inputs.shape[0]
    bt = 512
    grid = b // bt

    vrt = v.reshape(_F, _F, _D, _K).transpose(1, 0, 3, 2).reshape(_F, _GK, _D)
    vrt = jnp.concatenate(
        [vrt, w.reshape(_F, 1, _D),
         jnp.zeros((_F, _GKA - _GK - 1, _D), vrt.dtype)], axis=1)

    out = pl.pallas_call(
        _ffm_block,
        grid=(grid,),
        in_specs=[
            pl.BlockSpec((bt, _F * _D), lambda i: (i, 0)),
            pl.BlockSpec((_F, _GKA, _D), lambda i: (0, 0, 0)),
        ],
        out_specs=pl.BlockSpec((1, bt), lambda i: (0, i)),
        out_shape=jax.ShapeDtypeStruct((1, b), jnp.float32),
        scratch_shapes=[pltpu.VMEM((_F, _GKA, bt), jnp.float32)],
        compiler_params=pltpu.CompilerParams(
            dimension_semantics=("parallel",)),
    )(inputs, vrt)
    return out.reshape(b, 1)
